# baseline (speedup 1.0000x reference)
import functools

import jax
import jax.numpy as jnp
import numpy as np
from jax import lax
from jax.experimental import pallas as pl
from jax.experimental.pallas import tpu as pltpu
from jax.experimental.pallas import tpu_sc as plsc

V_DIM = 100000
EMB_DIM = 64
BATCH = 4096
CTX = 20

NUM_CORES = 2
NUM_SUBCORES = 16
NUM_WORKERS = NUM_CORES * NUM_SUBCORES
B_PER_W = BATCH // NUM_WORKERS
ROWS_PER_W = B_PER_W * CTX
CHUNK = 128
N_CHUNKS = ROWS_PER_W // CHUNK
LANES = 16


NBUF = 4


def _cbow_body(table_hbm, idx_hbm, dest_hbm, out_hbm,
               idx_v, dest_v, *scratch):
    bufs = scratch[:NBUF]
    acc_v, acc_sh = scratch[NBUF:NBUF + 2]
    gsems = scratch[NBUF + 2:2 * NBUF + 2]
    ssems = scratch[2 * NBUF + 2:]
    sid = lax.axis_index("s")
    wid = lax.axis_index("c") * NUM_SUBCORES + sid

    pltpu.sync_copy(idx_hbm.at[wid], idx_v)
    pltpu.sync_copy(dest_hbm.at[sid], dest_v)

    nbuf = NBUF
    copies = [None] * N_CHUNKS
    scat = [None] * N_CHUNKS
    for j in range(nbuf):
        copies[j] = pltpu.async_copy(
            table_hbm.at[idx_v.at[j]], bufs[j], gsems[j])

    @pl.loop(0, B_PER_W)
    def _(b):
        for c in range(EMB_DIM // LANES):
            acc_v[b, pl.ds(c * LANES, LANES)] = jnp.zeros((LANES,), jnp.float32)

    my_rows = pl.ds(sid * B_PER_W, B_PER_W)
    pltpu.sync_copy(acc_v, acc_sh.at[my_rows])

    for j in range(N_CHUNKS):
        copies[j].wait()
        scat[j] = pltpu.async_copy(
            bufs[j % nbuf], acc_sh.at[dest_v.at[j]], ssems[j % nbuf],
            add=True)
        if 1 <= j and j - 1 + nbuf < N_CHUNKS:
            scat[j - 1].wait()
            copies[j - 1 + nbuf] = pltpu.async_copy(
                table_hbm.at[idx_v.at[j - 1 + nbuf]], bufs[(j - 1) % nbuf],
                gsems[(j - 1) % nbuf])
    for j in range(max(0, N_CHUNKS - nbuf), N_CHUNKS):
        scat[j].wait()

    pltpu.sync_copy(acc_sh.at[my_rows], acc_v)
    scale = jnp.full((LANES,), 1.0 / CTX, jnp.float32)

    @pl.loop(0, B_PER_W)
    def _(b):
        for c in range(EMB_DIM // LANES):
            sl = pl.ds(c * LANES, LANES)
            acc_v[b, sl] = acc_v[b, sl] * scale

    pltpu.sync_copy(acc_v, out_hbm.at[pl.ds(wid * B_PER_W, B_PER_W)])


@jax.jit
def _cbow_sc(idx, embeddings, dest):
    mesh = plsc.VectorSubcoreMesh(core_axis_name="c", subcore_axis_name="s")
    kern = functools.partial(
        pl.kernel,
        out_type=jax.ShapeDtypeStruct((BATCH, EMB_DIM), jnp.float32),
        mesh=mesh,
        compiler_params=pltpu.CompilerParams(use_tc_tiling_on_sc=False),
        scratch_types=(
            [pltpu.VMEM((N_CHUNKS, CHUNK), jnp.int32),
             pltpu.VMEM((N_CHUNKS, CHUNK), jnp.int32)]
            + [pltpu.VMEM((CHUNK, EMB_DIM), jnp.float32)
               for _ in range(NBUF)]
            + [pltpu.VMEM((B_PER_W, EMB_DIM), jnp.float32),
               pltpu.VMEM_SHARED((NUM_SUBCORES * B_PER_W, EMB_DIM),
                                 jnp.float32)]
            + [pltpu.SemaphoreType.DMA for _ in range(2 * NBUF)]
        ),
    )(_cbow_body)
    return kern(embeddings, idx, dest)


_DEST = np.reshape(
    (np.arange(ROWS_PER_W, dtype=np.int32) // CTX)[None, :]
    + np.arange(NUM_SUBCORES, dtype=np.int32)[:, None] * B_PER_W,
    (NUM_SUBCORES, N_CHUNKS, CHUNK))


def kernel(x, embeddings):
    idx = x.astype(jnp.int32).reshape(NUM_WORKERS, N_CHUNKS, CHUNK)
    tbl = jax.lax.optimization_barrier(embeddings.reshape(V_DIM * EMB_DIM))
    tbl = tbl.reshape(V_DIM, EMB_DIM)
    return _cbow_sc(idx, tbl, jnp.asarray(_DEST))
